# Initial kernel scaffold; baseline (speedup 1.0000x reference)
import functools

import jax
import jax.numpy as jnp
from jax import lax
from jax.experimental import pallas as pl
from jax.experimental.pallas import tpu as pltpu
from jax.experimental.pallas import tpu_sc as plsc

N_NODES = 10000
N_EDGES = 320000
CHUNK = 128
N_CHUNKS = N_EDGES // CHUNK
NC = 2
NS = 16
NW = NC * NS
ROWS_PER_TILE = N_NODES // NS


def _spmm_body(feat, g_hbm, src_hbm, dst_hbm, w_hbm, z_hbm, out_hbm,
               src_v, dst_v, w_v, rows_v, accum, sem):
    cid = lax.axis_index("c")
    sid = lax.axis_index("s")
    wid = sid * NC + cid

    pltpu.sync_copy(z_hbm, accum.at[pl.ds(sid * ROWS_PER_TILE, ROWS_PER_TILE)])
    plsc.subcore_barrier()

    n_iters = (N_CHUNKS + NW - 1) // NW

    def chunk_body(k, carry):
        c = wid + NW * k

        @pl.when(c < N_CHUNKS)
        def _():
            pltpu.sync_copy(src_hbm.at[c], src_v)
            pltpu.sync_copy(dst_hbm.at[c], dst_v.at[0])
            pltpu.sync_copy(w_hbm.at[c], w_v)
            pltpu.async_copy(g_hbm.at[src_v], rows_v, sem).wait()

            def e_body(e, c2):
                ws = w_v[e]
                for j in range(feat // 16):
                    rows_v[e, pl.ds(j * 16, 16)] = rows_v[e, pl.ds(j * 16, 16)] * ws
                return c2

            lax.fori_loop(0, CHUNK, e_body, 0)
            pltpu.sync_copy(rows_v, accum.at[dst_v.at[0]], add=True)

        return carry

    lax.fori_loop(0, n_iters, chunk_body, 0)
    plsc.subcore_barrier()
    pltpu.sync_copy(accum.at[pl.ds(sid * ROWS_PER_TILE, ROWS_PER_TILE)],
                    out_hbm.at[cid, pl.ds(sid * ROWS_PER_TILE, ROWS_PER_TILE)])


def _spmm_sc(g, src2d, dst2d, w2d, zeros, feat):
    mesh = plsc.VectorSubcoreMesh(core_axis_name="c", subcore_axis_name="s")
    return pl.kernel(
        functools.partial(_spmm_body, feat),
        mesh=mesh,
        out_type=jax.ShapeDtypeStruct((NC, N_NODES, feat), jnp.float32),
        scratch_types=[
            pltpu.VMEM((CHUNK,), jnp.int32),
            pltpu.VMEM((1, CHUNK), jnp.int32),
            pltpu.VMEM((CHUNK,), jnp.float32),
            pltpu.VMEM((CHUNK, feat), jnp.float32),
            pltpu.VMEM_SHARED((N_NODES, feat), jnp.float32),
            pltpu.SemaphoreType.DMA,
        ],
    )(g, src2d, dst2d, w2d, zeros)


def _mm_body(p0_ref, p1_ref, w_ref, b_ref, o_ref, *, relu):
    s = p0_ref[...] + p1_ref[...]
    y = jnp.dot(s, w_ref[...], preferred_element_type=jnp.float32) + b_ref[...]
    if relu:
        y = jnp.maximum(y, 0.0)
    o_ref[...] = y


def _dense_tc(p, w, b, relu):
    din = w.shape[0]
    dout = w.shape[1]
    bm = 400
    grid = (N_NODES // bm,)
    return pl.pallas_call(
        functools.partial(_mm_body, relu=relu),
        grid=grid,
        in_specs=[
            pl.BlockSpec((bm, din), lambda i: (i, 0)),
            pl.BlockSpec((bm, din), lambda i: (i, 0)),
            pl.BlockSpec((din, dout), lambda i: (0, 0)),
            pl.BlockSpec((1, dout), lambda i: (0, 0)),
        ],
        out_specs=pl.BlockSpec((bm, dout), lambda i: (i, 0)),
        out_shape=jax.ShapeDtypeStruct((N_NODES, dout), jnp.float32),
    )(p[0], p[1], w, b)


def kernel(x, edge_index, edge_weight, Ws, bs):
    src2d = edge_index[0].reshape(N_CHUNKS, CHUNK)
    dst2d = edge_index[1].reshape(N_CHUNKS, CHUNK)
    w2d = edge_weight.reshape(N_CHUNKS, CHUNK)
    zeros = jnp.zeros((ROWS_PER_TILE, 128), dtype=jnp.float32)

    h = x
    n_layers = len(Ws)
    for i in range(n_layers):
        p = _spmm_sc(h, src2d, dst2d, w2d, zeros, feat=h.shape[1])
        h = _dense_tc(p, Ws[i], bs[i].reshape(1, -1), relu=(i != n_layers - 1))
    return h


# speedup vs baseline: 4.7394x; 4.7394x over previous
import functools

import jax
import jax.numpy as jnp
from jax import lax
from jax.experimental import pallas as pl
from jax.experimental.pallas import tpu as pltpu
from jax.experimental.pallas import tpu_sc as plsc

N_NODES = 10000
N_EDGES = 320000
CHUNK = 128
N_CHUNKS = N_EDGES // CHUNK
NC = 2
NS = 16
NW = NC * NS
ROWS_A = 624
ROWS_LAST = N_NODES - 15 * ROWS_A


def _spmm_body(feat, g_hbm, src_hbm, dst_hbm, w_hbm, z_hbm, out_hbm,
               src_v, dst_v, w_v, rows_v, accum, sem):
    cid = lax.axis_index("c")
    sid = lax.axis_index("s")
    wid = sid * NC + cid

    @pl.when(sid < 15)
    def _():
        pltpu.sync_copy(z_hbm.at[pl.ds(0, ROWS_A)],
                        accum.at[pl.ds(sid * ROWS_A, ROWS_A)])

    @pl.when(sid == 15)
    def _():
        pltpu.sync_copy(z_hbm, accum.at[pl.ds(15 * ROWS_A, ROWS_LAST)])

    plsc.subcore_barrier()

    n_iters = (N_CHUNKS + NW - 1) // NW

    def chunk_body(k, carry):
        c = wid + NW * k

        @pl.when(c < N_CHUNKS)
        def _():
            pltpu.sync_copy(src_hbm.at[c], src_v)
            pltpu.sync_copy(dst_hbm.at[c], dst_v.at[0])
            pltpu.sync_copy(w_hbm.at[c], w_v)
            pltpu.async_copy(g_hbm.at[src_v], rows_v, sem).wait()

            def blk_body(b, c2):
                wv = w_v[pl.ds(b * 16, 16)]
                for t in range(16):
                    ws = wv[t]
                    e = b * 16 + t
                    for j in range(feat // 16):
                        rows_v[e, pl.ds(j * 16, 16)] = (
                            rows_v[e, pl.ds(j * 16, 16)] * ws)
                return c2

            lax.fori_loop(0, CHUNK // 16, blk_body, 0)
            pltpu.sync_copy(rows_v, accum.at[dst_v.at[0]], add=True)

        return carry

    lax.fori_loop(0, n_iters, chunk_body, 0)
    plsc.subcore_barrier()

    @pl.when(sid < 15)
    def _():
        pltpu.sync_copy(accum.at[pl.ds(sid * ROWS_A, ROWS_A)],
                        out_hbm.at[cid, pl.ds(sid * ROWS_A, ROWS_A)])

    @pl.when(sid == 15)
    def _():
        pltpu.sync_copy(accum.at[pl.ds(15 * ROWS_A, ROWS_LAST)],
                        out_hbm.at[cid, pl.ds(15 * ROWS_A, ROWS_LAST)])


def _spmm_sc(g, src2d, dst2d, w2d, zeros, feat):
    mesh = plsc.VectorSubcoreMesh(core_axis_name="c", subcore_axis_name="s")
    return pl.kernel(
        functools.partial(_spmm_body, feat),
        mesh=mesh,
        out_type=jax.ShapeDtypeStruct((NC, N_NODES, feat), jnp.float32),
        scratch_types=[
            pltpu.VMEM((CHUNK,), jnp.int32),
            pltpu.VMEM((1, CHUNK), jnp.int32),
            pltpu.VMEM((CHUNK,), jnp.float32),
            pltpu.VMEM((CHUNK, feat), jnp.float32),
            pltpu.VMEM_SHARED((N_NODES, feat), jnp.float32),
            pltpu.SemaphoreType.DMA,
        ],
    )(g, src2d, dst2d, w2d, zeros)


def _mm_body(p0_ref, p1_ref, w_ref, b_ref, o_ref, *, relu):
    s = p0_ref[...] + p1_ref[...]
    y = jnp.dot(s, w_ref[...], preferred_element_type=jnp.float32) + b_ref[...]
    if relu:
        y = jnp.maximum(y, 0.0)
    o_ref[...] = y


def _dense_tc(p, w, b, relu):
    din = w.shape[0]
    dout = w.shape[1]
    bm = 400
    grid = (N_NODES // bm,)
    return pl.pallas_call(
        functools.partial(_mm_body, relu=relu),
        grid=grid,
        in_specs=[
            pl.BlockSpec((bm, din), lambda i: (i, 0)),
            pl.BlockSpec((bm, din), lambda i: (i, 0)),
            pl.BlockSpec((din, dout), lambda i: (0, 0)),
            pl.BlockSpec((1, dout), lambda i: (0, 0)),
        ],
        out_specs=pl.BlockSpec((bm, dout), lambda i: (i, 0)),
        out_shape=jax.ShapeDtypeStruct((N_NODES, dout), jnp.float32),
    )(p[0], p[1], w, b)


def kernel(x, edge_index, edge_weight, Ws, bs):
    src2d = edge_index[0].reshape(N_CHUNKS, CHUNK)
    dst2d = edge_index[1].reshape(N_CHUNKS, CHUNK)
    w2d = edge_weight.reshape(N_CHUNKS, CHUNK)
    zeros = jnp.zeros((ROWS_LAST, 128), dtype=jnp.float32)

    h = x
    n_layers = len(Ws)
    for i in range(n_layers):
        p = _spmm_sc(h, src2d, dst2d, w2d, zeros, feat=h.shape[1])
        h = _dense_tc(p, Ws[i], bs[i].reshape(1, -1), relu=(i != n_layers - 1))
    return h


# speedup vs baseline: 10.6427x; 2.2456x over previous
import functools

import jax
import jax.numpy as jnp
from jax import lax
from jax.experimental import pallas as pl
from jax.experimental.pallas import tpu as pltpu
from jax.experimental.pallas import tpu_sc as plsc

N_NODES = 10000
N_EDGES = 320000
CHUNK = 128
N_CHUNKS = N_EDGES // CHUNK
NC = 2
NS = 16
NW = NC * NS
NBUF = 3
ROWS_A = 624
ROWS_LAST = N_NODES - 15 * ROWS_A


def _spmm_body(feat, g_hbm, e_hbm, w_hbm, z_hbm, out_hbm,
               eb0, eb1, eb2, wb0, wb1, wb2, rb0, rb1, rb2, accum,
               s0, s1, s2):
    ebufs = (eb0, eb1, eb2)
    wbufs = (wb0, wb1, wb2)
    rbufs = (rb0, rb1, rb2)
    sems = (s0, s1, s2)
    cid = lax.axis_index("c")
    sid = lax.axis_index("s")
    wid = sid * NC + cid

    @pl.when(sid < 15)
    def _():
        pltpu.sync_copy(z_hbm.at[pl.ds(0, ROWS_A)],
                        accum.at[pl.ds(sid * ROWS_A, ROWS_A)])

    @pl.when(sid == 15)
    def _():
        pltpu.sync_copy(z_hbm, accum.at[pl.ds(15 * ROWS_A, ROWS_LAST)])

    plsc.subcore_barrier()

    n_valid = (N_CHUNKS - wid + NW - 1) // NW

    def idx_start(k, b):
        @pl.when(k < n_valid)
        def _():
            c = wid + NW * k
            pltpu.async_copy(e_hbm.at[c], ebufs[b], sems[b])
            pltpu.async_copy(w_hbm.at[c], wbufs[b], sems[b])

    def idx_wait(k, b):
        @pl.when(k < n_valid)
        def _():
            pltpu.make_async_copy(e_hbm.at[0], ebufs[b], sems[b]).wait()
            pltpu.make_async_copy(w_hbm.at[0], wbufs[b], sems[b]).wait()

    def gather_start(k, b):
        @pl.when(k < n_valid)
        def _():
            pltpu.async_copy(g_hbm.at[ebufs[b].at[0]], rbufs[b], sems[b])

    def gather_wait(k, b):
        @pl.when(k < n_valid)
        def _():
            pltpu.make_async_copy(g_hbm.at[ebufs[b].at[0]], rbufs[b],
                                  sems[b]).wait()

    def scale(k, b):
        @pl.when(k < n_valid)
        def _():
            rows_v = rbufs[b]
            eb = ebufs[b]

            wb = wbufs[b]

            def blk_body(blk, c2):
                wv = wb[pl.ds(blk * 16, 16)]
                for t in range(16):
                    ws = wv[t]
                    e = blk * 16 + t
                    for j in range(feat // 16):
                        rows_v[e, pl.ds(j * 16, 16)] = (
                            rows_v[e, pl.ds(j * 16, 16)] * ws)
                return c2

            lax.fori_loop(0, CHUNK // 16, blk_body, 0)

    def scatter_start(k, b):
        @pl.when(k < n_valid)
        def _():
            pltpu.async_copy(rbufs[b], accum.at[ebufs[b].at[1]], sems[b],
                             add=True)

    def scatter_wait(k, b):
        @pl.when(jnp.logical_and(k >= 0, k < n_valid))
        def _():
            pltpu.make_async_copy(rbufs[b], accum.at[ebufs[b].at[1]],
                                  sems[b]).wait()

    idx_start(0, 0)
    idx_wait(0, 0)
    gather_start(0, 0)
    idx_start(1, 1)

    n_outer = (N_CHUNKS + NW - 1) // NW + 2
    assert n_outer % NBUF == 0

    def outer_body(kk, carry):
        for b in range(NBUF):
            k = kk * NBUF + b
            scatter_wait(k - 1, (b + 2) % NBUF)
            idx_start(k + 2, (b + 2) % NBUF)
            idx_wait(k + 1, (b + 1) % NBUF)
            gather_start(k + 1, (b + 1) % NBUF)
            gather_wait(k, b)
            scale(k, b)
            scatter_start(k, b)
        return carry

    lax.fori_loop(0, n_outer // NBUF, outer_body, 0)

    plsc.subcore_barrier()

    @pl.when(sid < 15)
    def _():
        pltpu.sync_copy(accum.at[pl.ds(sid * ROWS_A, ROWS_A)],
                        out_hbm.at[cid, pl.ds(sid * ROWS_A, ROWS_A)])

    @pl.when(sid == 15)
    def _():
        pltpu.sync_copy(accum.at[pl.ds(15 * ROWS_A, ROWS_LAST)],
                        out_hbm.at[cid, pl.ds(15 * ROWS_A, ROWS_LAST)])


def _spmm_sc(g, edata, w2d, zeros, feat):
    mesh = plsc.VectorSubcoreMesh(core_axis_name="c", subcore_axis_name="s")
    return pl.kernel(
        functools.partial(_spmm_body, feat),
        mesh=mesh,
        out_type=jax.ShapeDtypeStruct((NC, N_NODES, feat), jnp.float32),
        scratch_types=[
            pltpu.VMEM((2, CHUNK), jnp.int32),
            pltpu.VMEM((2, CHUNK), jnp.int32),
            pltpu.VMEM((2, CHUNK), jnp.int32),
            pltpu.VMEM((CHUNK,), jnp.float32),
            pltpu.VMEM((CHUNK,), jnp.float32),
            pltpu.VMEM((CHUNK,), jnp.float32),
            pltpu.VMEM((CHUNK, feat), jnp.float32),
            pltpu.VMEM((CHUNK, feat), jnp.float32),
            pltpu.VMEM((CHUNK, feat), jnp.float32),
            pltpu.VMEM_SHARED((N_NODES, feat), jnp.float32),
            pltpu.SemaphoreType.DMA,
            pltpu.SemaphoreType.DMA,
            pltpu.SemaphoreType.DMA,
        ],
    )(g, edata, w2d, zeros)


def _mm_body(p0_ref, p1_ref, w_ref, b_ref, o_ref, *, relu):
    s = p0_ref[...] + p1_ref[...]
    y = jnp.dot(s, w_ref[...], preferred_element_type=jnp.float32) + b_ref[...]
    if relu:
        y = jnp.maximum(y, 0.0)
    o_ref[...] = y


def _dense_tc(p, w, b, relu):
    din = w.shape[0]
    dout = w.shape[1]
    bm = 400
    grid = (N_NODES // bm,)
    return pl.pallas_call(
        functools.partial(_mm_body, relu=relu),
        grid=grid,
        in_specs=[
            pl.BlockSpec((bm, din), lambda i: (i, 0)),
            pl.BlockSpec((bm, din), lambda i: (i, 0)),
            pl.BlockSpec((din, dout), lambda i: (0, 0)),
            pl.BlockSpec((1, dout), lambda i: (0, 0)),
        ],
        out_specs=pl.BlockSpec((bm, dout), lambda i: (i, 0)),
        out_shape=jax.ShapeDtypeStruct((N_NODES, dout), jnp.float32),
    )(p[0], p[1], w, b)


def kernel(x, edge_index, edge_weight, Ws, bs):
    edata = edge_index.reshape(2, N_CHUNKS, CHUNK).transpose(1, 0, 2)
    w2d = edge_weight.reshape(N_CHUNKS, CHUNK)
    zeros = jnp.zeros((ROWS_LAST, 128), dtype=jnp.float32)

    h = x
    n_layers = len(Ws)
    for i in range(n_layers):
        p = _spmm_sc(h, edata, w2d, zeros, feat=h.shape[1])
        h = _dense_tc(p, Ws[i], bs[i].reshape(1, -1), relu=(i != n_layers - 1))
    return h


# speedup vs baseline: 10.9919x; 1.0328x over previous
import functools

import jax
import jax.numpy as jnp
from jax import lax
from jax.experimental import pallas as pl
from jax.experimental.pallas import tpu as pltpu
from jax.experimental.pallas import tpu_sc as plsc

N_NODES = 10000
N_EDGES = 320000
CHUNK = 96
N_CHUNKS = -(-N_EDGES // CHUNK)
E_PAD = N_CHUNKS * CHUNK
NC = 2
NS = 16
NW = NC * NS
NBUF = 4
ROWS_A = 624
ROWS_LAST = N_NODES - 15 * ROWS_A


def _spmm_body(feat, g_hbm, e_hbm, w_hbm, z_hbm, out_hbm,
               eb0, eb1, eb2, eb3, wb0, wb1, wb2, wb3,
               rb0, rb1, rb2, rb3, accum, s0, s1, s2, s3):
    ebufs = (eb0, eb1, eb2, eb3)
    wbufs = (wb0, wb1, wb2, wb3)
    rbufs = (rb0, rb1, rb2, rb3)
    sems = (s0, s1, s2, s3)
    cid = lax.axis_index("c")
    sid = lax.axis_index("s")
    wid = sid * NC + cid

    @pl.when(sid < 15)
    def _():
        pltpu.sync_copy(z_hbm.at[pl.ds(0, ROWS_A)],
                        accum.at[pl.ds(sid * ROWS_A, ROWS_A)])

    @pl.when(sid == 15)
    def _():
        pltpu.sync_copy(z_hbm, accum.at[pl.ds(15 * ROWS_A, ROWS_LAST)])

    plsc.subcore_barrier()

    n_valid = (N_CHUNKS - wid + NW - 1) // NW

    def idx_start(k, b):
        @pl.when(k < n_valid)
        def _():
            c = wid + NW * k
            pltpu.async_copy(e_hbm.at[c], ebufs[b], sems[b])
            pltpu.async_copy(w_hbm.at[c], wbufs[b], sems[b])

    def idx_wait(k, b):
        @pl.when(k < n_valid)
        def _():
            pltpu.make_async_copy(e_hbm.at[0], ebufs[b], sems[b]).wait()
            pltpu.make_async_copy(w_hbm.at[0], wbufs[b], sems[b]).wait()

    def gather_start(k, b):
        @pl.when(k < n_valid)
        def _():
            pltpu.async_copy(g_hbm.at[ebufs[b].at[0]], rbufs[b], sems[b])

    def gather_wait(k, b):
        @pl.when(k < n_valid)
        def _():
            pltpu.make_async_copy(g_hbm.at[ebufs[b].at[0]], rbufs[b],
                                  sems[b]).wait()

    def scale(k, b):
        @pl.when(k < n_valid)
        def _():
            rows_v = rbufs[b]
            eb = ebufs[b]

            wb = wbufs[b]

            def blk_body(blk, c2):
                wv = wb[pl.ds(blk * 16, 16)]
                for t in range(16):
                    ws = wv[t]
                    e = blk * 16 + t
                    for j in range(feat // 16):
                        rows_v[e, pl.ds(j * 16, 16)] = (
                            rows_v[e, pl.ds(j * 16, 16)] * ws)
                return c2

            lax.fori_loop(0, CHUNK // 16, blk_body, 0)

    def scatter_start(k, b):
        @pl.when(k < n_valid)
        def _():
            pltpu.async_copy(rbufs[b], accum.at[ebufs[b].at[1]], sems[b],
                             add=True)

    def scatter_wait(k, b):
        @pl.when(jnp.logical_and(k >= 0, k < n_valid))
        def _():
            pltpu.make_async_copy(rbufs[b], accum.at[ebufs[b].at[1]],
                                  sems[b]).wait()

    idx_start(0, 0)
    idx_wait(0, 0)
    gather_start(0, 0)
    idx_start(1, 1)

    n_outer = (((N_CHUNKS + NW - 1) // NW + 2) + NBUF - 1) // NBUF * NBUF

    def outer_body(kk, carry):
        for b in range(NBUF):
            k = kk * NBUF + b
            scatter_wait(k - 2, (b + 2) % NBUF)
            idx_start(k + 2, (b + 2) % NBUF)
            idx_wait(k + 1, (b + 1) % NBUF)
            gather_start(k + 1, (b + 1) % NBUF)
            gather_wait(k, b)
            scale(k, b)
            scatter_start(k, b)
        return carry

    lax.fori_loop(0, n_outer // NBUF, outer_body, 0)

    plsc.subcore_barrier()

    @pl.when(sid < 15)
    def _():
        pltpu.sync_copy(accum.at[pl.ds(sid * ROWS_A, ROWS_A)],
                        out_hbm.at[cid, pl.ds(sid * ROWS_A, ROWS_A)])

    @pl.when(sid == 15)
    def _():
        pltpu.sync_copy(accum.at[pl.ds(15 * ROWS_A, ROWS_LAST)],
                        out_hbm.at[cid, pl.ds(15 * ROWS_A, ROWS_LAST)])


def _spmm_sc(g, edata, w2d, zeros, feat):
    mesh = plsc.VectorSubcoreMesh(core_axis_name="c", subcore_axis_name="s")
    return pl.kernel(
        functools.partial(_spmm_body, feat),
        mesh=mesh,
        out_type=jax.ShapeDtypeStruct((NC, N_NODES, feat), jnp.float32),
        scratch_types=(
            [pltpu.VMEM((2, CHUNK), jnp.int32)] * NBUF
            + [pltpu.VMEM((CHUNK,), jnp.float32)] * NBUF
            + [pltpu.VMEM((CHUNK, feat), jnp.float32)] * NBUF
            + [pltpu.VMEM_SHARED((N_NODES, feat), jnp.float32)]
            + [pltpu.SemaphoreType.DMA] * NBUF
        ),
    )(g, edata, w2d, zeros)


def _mm_body(p0_ref, p1_ref, w_ref, b_ref, o_ref, *, relu):
    s = p0_ref[...] + p1_ref[...]
    y = jnp.dot(s, w_ref[...], preferred_element_type=jnp.float32) + b_ref[...]
    if relu:
        y = jnp.maximum(y, 0.0)
    o_ref[...] = y


def _dense_tc(p, w, b, relu):
    din = w.shape[0]
    dout = w.shape[1]
    bm = 400
    grid = (N_NODES // bm,)
    return pl.pallas_call(
        functools.partial(_mm_body, relu=relu),
        grid=grid,
        in_specs=[
            pl.BlockSpec((bm, din), lambda i: (i, 0)),
            pl.BlockSpec((bm, din), lambda i: (i, 0)),
            pl.BlockSpec((din, dout), lambda i: (0, 0)),
            pl.BlockSpec((1, dout), lambda i: (0, 0)),
        ],
        out_specs=pl.BlockSpec((bm, dout), lambda i: (i, 0)),
        out_shape=jax.ShapeDtypeStruct((N_NODES, dout), jnp.float32),
    )(p[0], p[1], w, b)


def kernel(x, edge_index, edge_weight, Ws, bs):
    pad = E_PAD - N_EDGES
    ei = jnp.pad(edge_index, ((0, 0), (0, pad)))
    ew = jnp.pad(edge_weight, (0, pad))
    edata = ei.reshape(2, N_CHUNKS, CHUNK).transpose(1, 0, 2)
    w2d = ew.reshape(N_CHUNKS, CHUNK)
    zeros = jnp.zeros((ROWS_LAST, 128), dtype=jnp.float32)

    h = x
    n_layers = len(Ws)
    for i in range(n_layers):
        p = _spmm_sc(h, edata, w2d, zeros, feat=h.shape[1])
        h = _dense_tc(p, Ws[i], bs[i].reshape(1, -1), relu=(i != n_layers - 1))
    return h
